# Initial kernel scaffold; baseline (speedup 1.0000x reference)
import jax
import jax.numpy as jnp
from jax.experimental import pallas as pl


def kernel(X, A, S, kernel):
    raise NotImplementedError("write your pallas kernel here")


# speedup vs baseline: 1.6828x; 1.6828x over previous
import functools

import jax
import jax.numpy as jnp
from jax import lax
from jax.experimental import pallas as pl
from jax.experimental.pallas import tpu as pltpu
from jax.experimental.pallas import tpu_sc as plsc

N = 4096
F = 512
KP = 1024

_HI = jax.lax.Precision.HIGHEST


def _feat_body(x_ref, w_ref, s_ref, feat_ref, y_ref):
    w = w_ref[...]
    nrm = jax.lax.rsqrt(jnp.maximum(jnp.sum(w * w), 1e-12))
    y = jnp.dot(x_ref[...], w * nrm,
                preferred_element_type=jnp.float32)
    y_ref[...] = y
    feat_ref[:, :F] = x_ref[...] * jnp.tanh(y)
    sbc = lax.bitcast_convert_type(s_ref[...], jnp.float32)
    feat_ref[:, F:] = jnp.broadcast_to(sbc, (N, 128))


def _features(X, w, S):
    return pl.pallas_call(
        _feat_body,
        out_shape=(
            jax.ShapeDtypeStruct((N, F + 128), jnp.float32),
            jax.ShapeDtypeStruct((N, 1), jnp.float32),
        ),
    )(X, w, jnp.reshape(S, (N, 1)))


def _select_body(scol_ref, srow_ref, idx_ref, mask_ref):
    srow = srow_ref[...]
    cw = 512
    for ci in range(N // cw):
        sc = scol_ref[pl.ds(ci * cw, cw), :]
        jj = lax.broadcasted_iota(jnp.int32, (cw, N), 1)
        ii = lax.broadcasted_iota(jnp.int32, (cw, N), 0) + ci * cw
        before = (srow > sc) | ((srow == sc) & (jj < ii))
        rank = jnp.sum(before.astype(jnp.float32), axis=1, keepdims=True)
        mask_ref[pl.ds(ci * cw, cw), :] = (rank < KP).astype(jnp.float32)
    crow = jnp.zeros((1, N), jnp.float32)
    for ci in range(N // cw):
        mc = mask_ref[pl.ds(ci * cw, cw), :]
        jj = lax.broadcasted_iota(jnp.int32, (cw, N), 0) + ci * cw
        ii = lax.broadcasted_iota(jnp.int32, (cw, N), 1)
        crow = crow + jnp.sum(mc * (jj <= ii).astype(jnp.float32),
                              axis=0, keepdims=True)
    for pi in range(KP // cw):
        pp = (lax.broadcasted_iota(jnp.int32, (cw, N), 0) + pi * cw
              ).astype(jnp.float32)
        cnt = jnp.sum((crow <= pp).astype(jnp.float32), axis=1, keepdims=True)
        idx_ref[pl.ds(pi * cw, cw), :] = cnt.astype(jnp.int32)


def _select(y):
    s_col = y
    s_row = jnp.reshape(y, (1, N))
    return pl.pallas_call(
        _select_body,
        out_shape=jax.ShapeDtypeStruct((KP, 1), jnp.int32),
        scratch_shapes=[pltpu.VMEM((N, 1), jnp.float32)],
    )(s_col, s_row)


_TB = 512


def _tr_body(a_ref, o_ref):
    o_ref[...] = a_ref[...].T


def _transpose(A):
    g = N // _TB
    return pl.pallas_call(
        _tr_body,
        grid=(g, g),
        in_specs=[pl.BlockSpec((_TB, _TB), lambda i, j: (j, i))],
        out_specs=pl.BlockSpec((_TB, _TB), lambda i, j: (i, j)),
        out_shape=jax.ShapeDtypeStruct((N, N), jnp.float32),
    )(A)


_NC = 2
_NS = 16
_NW = _NC * _NS
_BPW = KP // _NW
_AC = 8


def _sc_gather_body(feat_hbm, a_hbm, at_hbm, idx_hbm, idx2_hbm,
                    xp_out, ar_out, atr_out,
                    idx_v, idxc_v, xbuf, abuf, sem):
    wid = lax.axis_index("s") * _NC + lax.axis_index("c")
    base = wid * _BPW
    pltpu.sync_copy(idx_hbm.at[pl.ds(base, _BPW)], idx_v)
    pltpu.sync_copy(idx2_hbm.at[pl.ds(wid * (_BPW // _AC), _BPW // _AC)],
                    idxc_v)
    pltpu.async_copy(feat_hbm.at[idx_v], xbuf, sem).wait()
    pltpu.sync_copy(xbuf, xp_out.at[pl.ds(base, _BPW)])
    for c in range(_BPW // _AC):
        pltpu.async_copy(a_hbm.at[idxc_v.at[c]], abuf, sem).wait()
        pltpu.sync_copy(abuf, ar_out.at[pl.ds(base + c * _AC, _AC)])
    for c in range(_BPW // _AC):
        pltpu.async_copy(at_hbm.at[idxc_v.at[c]], abuf, sem).wait()
        pltpu.sync_copy(abuf, atr_out.at[pl.ds(base + c * _AC, _AC)])


def _sc_gather(feat, A, At, idx):
    mesh = plsc.VectorSubcoreMesh(core_axis_name="c", subcore_axis_name="s")
    idx2 = jnp.reshape(idx, (KP // _AC, _AC))
    run = functools.partial(
        pl.kernel,
        mesh=mesh,
        out_type=[
            jax.ShapeDtypeStruct((KP, F + 128), jnp.float32),
            jax.ShapeDtypeStruct((KP, N), jnp.float32),
            jax.ShapeDtypeStruct((KP, N), jnp.float32),
        ],
        scratch_types=[
            pltpu.VMEM((_BPW,), jnp.int32),
            pltpu.VMEM((_BPW // _AC, _AC), jnp.int32),
            pltpu.VMEM((_BPW, F + 128), jnp.float32),
            pltpu.VMEM((_AC, N), jnp.float32),
            pltpu.SemaphoreType.DMA,
        ],
    )(_sc_gather_body)
    return run(feat, A, At, idx, idx2)


_MB = 512


def _mm_body(ar_ref, atr_ref, o_ref):
    o_ref[...] = lax.dot_general(
        ar_ref[...], atr_ref[...], (((1,), (1,)), ((), ())),
        preferred_element_type=jnp.float32)


def _pool_matmul(Ar, Atr):
    g = KP // _MB
    return pl.pallas_call(
        _mm_body,
        grid=(g, g),
        in_specs=[
            pl.BlockSpec((_MB, N), lambda i, j: (i, 0)),
            pl.BlockSpec((_MB, N), lambda i, j: (j, 0)),
        ],
        out_specs=pl.BlockSpec((_MB, _MB), lambda i, j: (i, j)),
        out_shape=jax.ShapeDtypeStruct((KP, KP), jnp.float32),
    )(Ar, Atr)


def kernel(X, A, S, kernel):
    feat, y = _features(X, kernel, S)
    idx2d = _select(y)
    idx = jnp.reshape(idx2d, (KP,))
    At = _transpose(A)
    G, Ar, Atr = _sc_gather(feat, A, At, idx)
    Ap = _pool_matmul(Ar, Atr)
    Xp = G[:, :F]
    Sp = lax.bitcast_convert_type(G[:, F], jnp.int32)
    return Xp, Ap, Sp
